# baseline (speedup 1.0000x reference)
import functools

import jax
import jax.numpy as jnp
from jax import lax
from jax.experimental import pallas as pl
from jax.experimental.pallas import tpu as pltpu
from jax.experimental.pallas import tpu_sc as plsc

BATCH = 16384
D_ITEM = 64
D_AGE = 16
D_GENDER = 8
AGE_ROWS_PAD = 128
GEN_ROWS_PAD = 8
IDX_CHUNK = 128


def _gather_item_rows(item_table, idx):
    info = plsc.get_sparse_core_info()
    nc, ns = info.num_cores, info.num_subcores
    nw = nc * ns
    b_per_w = BATCH // nw
    n_chunks = b_per_w // IDX_CHUNK
    idx3 = idx.astype(jnp.int32).reshape(nw, n_chunks, IDX_CHUNK)
    mesh = plsc.VectorSubcoreMesh(core_axis_name="c", subcore_axis_name="s")

    @functools.partial(
        pl.kernel,
        mesh=mesh,
        compiler_params=pltpu.CompilerParams(use_tc_tiling_on_sc=False),
        out_type=jax.ShapeDtypeStruct((BATCH, D_ITEM), jnp.float32),
        scratch_types=[
            pltpu.VMEM((n_chunks, IDX_CHUNK), jnp.int32),
            pltpu.VMEM((n_chunks, IDX_CHUNK, D_ITEM), jnp.float32),
            pltpu.SemaphoreType.DMA,
        ],
    )
    def gather_k(table_hbm, idx_hbm, out_hbm, idx_v, rows_v, sem):
        wid = lax.axis_index("s") * nc + lax.axis_index("c")
        base = wid * b_per_w
        pltpu.sync_copy(idx_hbm.at[wid], idx_v)
        copies = [
            pltpu.async_copy(table_hbm.at[idx_v.at[j]], rows_v.at[j], sem)
            for j in range(n_chunks)
        ]
        for cp in copies:
            cp.wait()
        for j in range(n_chunks):
            pltpu.sync_copy(
                rows_v.at[j], out_hbm.at[pl.ds(base + j * IDX_CHUNK, IDX_CHUNK)]
            )

    return gather_k(item_table, idx3)


def _mlp_block(item_ref, age_ref, gen_ref, aget_ref, gent_ref,
               w1a_ref, w1b_ref, w1c_ref, b1_ref, w2_ref, b2_ref,
               w3_ref, b3_ref, out_ref):
    bb = item_ref.shape[0]
    f32 = jnp.float32
    age_oh = (age_ref[...] ==
              lax.broadcasted_iota(jnp.int32, (bb, AGE_ROWS_PAD), 1)).astype(f32)
    gen_oh = (gen_ref[...] ==
              lax.broadcasted_iota(jnp.int32, (bb, GEN_ROWS_PAD), 1)).astype(f32)
    age_emb = jnp.dot(age_oh, aget_ref[...], preferred_element_type=f32)
    gen_emb = jnp.dot(gen_oh, gent_ref[...], preferred_element_type=f32)
    h = (jnp.dot(item_ref[...], w1a_ref[...], preferred_element_type=f32)
         + jnp.dot(age_emb, w1b_ref[...], preferred_element_type=f32)
         + jnp.dot(gen_emb, w1c_ref[...], preferred_element_type=f32)
         + b1_ref[...])
    h = jnp.maximum(h, 0.0)
    h = jnp.maximum(jnp.dot(h, w2_ref[...], preferred_element_type=f32)
                    + b2_ref[...], 0.0)
    out_ref[...] = jnp.dot(h, w3_ref[...], preferred_element_type=f32) + b3_ref[...]


def kernel(query_itemid, query_item_age, query_item_gender,
           item_table, age_table, gender_table,
           W1, b1, W2, b2, W3, b3):
    item_emb = _gather_item_rows(item_table, query_itemid)

    age2 = query_item_age.astype(jnp.int32).reshape(BATCH, 1)
    gen2 = query_item_gender.astype(jnp.int32).reshape(BATCH, 1)
    aget = jnp.pad(age_table, ((0, AGE_ROWS_PAD - age_table.shape[0]), (0, 0)))
    gent = jnp.pad(gender_table, ((0, GEN_ROWS_PAD - gender_table.shape[0]), (0, 0)))
    w1a = W1[:D_ITEM]
    w1b = W1[D_ITEM:D_ITEM + D_AGE]
    w1c = W1[D_ITEM + D_AGE:]
    b1r = b1.reshape(1, -1)
    b2r = b2.reshape(1, -1)
    b3r = b3.reshape(1, -1)

    bb = 2048
    grid = (BATCH // bb,)
    const = lambda i: (0, 0)
    out = pl.pallas_call(
        _mlp_block,
        grid=grid,
        in_specs=[
            pl.BlockSpec((bb, D_ITEM), lambda i: (i, 0)),
            pl.BlockSpec((bb, 1), lambda i: (i, 0)),
            pl.BlockSpec((bb, 1), lambda i: (i, 0)),
            pl.BlockSpec((AGE_ROWS_PAD, D_AGE), const),
            pl.BlockSpec((GEN_ROWS_PAD, D_GENDER), const),
            pl.BlockSpec((D_ITEM, 256), const),
            pl.BlockSpec((D_AGE, 256), const),
            pl.BlockSpec((D_GENDER, 256), const),
            pl.BlockSpec((1, 256), const),
            pl.BlockSpec((256, 128), const),
            pl.BlockSpec((1, 128), const),
            pl.BlockSpec((128, 64), const),
            pl.BlockSpec((1, 64), const),
        ],
        out_specs=pl.BlockSpec((bb, 64), lambda i: (i, 0)),
        out_shape=jax.ShapeDtypeStruct((BATCH, 64), jnp.float32),
    )(item_emb, age2, gen2, aget, gent, w1a, w1b, w1c, b1r, W2, b2r, W3, b3r)
    return out


# speedup vs baseline: 1.0459x; 1.0459x over previous
import functools

import jax
import jax.numpy as jnp
from jax import lax
from jax.experimental import pallas as pl
from jax.experimental.pallas import tpu as pltpu
from jax.experimental.pallas import tpu_sc as plsc

BATCH = 16384
VOCAB = 1000000
D_ITEM = 64
D_AGE = 16
D_GENDER = 8
AGE_ROWS_PAD = 128
GEN_ROWS_PAD = 8
IDX_CHUNK = 128


def _gather_item_rows(item_table, idx):
    info = plsc.get_sparse_core_info()
    nc = info.num_cores
    b_per_c = BATCH // nc
    ch = 1024
    n_chunks = b_per_c // ch

    idx1 = idx.astype(jnp.int32)
    mesh = plsc.ScalarSubcoreMesh(axis_name="c", num_cores=nc)

    @functools.partial(
        pl.kernel,
        mesh=mesh,
        out_type=jax.ShapeDtypeStruct((BATCH, D_ITEM), jnp.float32),
        scratch_types=[
            pltpu.SMEM((ch,), jnp.int32),
            pltpu.SemaphoreType.DMA,
            pltpu.SemaphoreType.DMA,
        ],
    )
    def gather_k(table_hbm, idx_hbm, out_hbm, idx_s, sem, isem):
        cid = lax.axis_index("c")
        base = cid * b_per_c

        def chunk_body(c, _):
            pltpu.async_copy(
                idx_hbm.at[pl.ds(base + c * ch, ch)], idx_s, isem
            ).wait()

            def fire(j, _):
                row = idx_s[j]
                pltpu.async_copy(
                    table_hbm.at[pl.ds(row, 1)],
                    out_hbm.at[pl.ds(base + c * ch + j, 1)],
                    sem,
                )
                return 0

            lax.fori_loop(0, ch, fire, 0)
            return 0

        lax.fori_loop(0, n_chunks, chunk_body, 0)
        pltpu.make_async_copy(
            table_hbm.at[pl.ds(0, b_per_c)],
            out_hbm.at[pl.ds(base, b_per_c)],
            sem,
        ).wait()

    return gather_k(item_table, idx1)


def _mlp_block(item_ref, age_ref, gen_ref, aget_ref, gent_ref,
               w1a_ref, w1b_ref, w1c_ref, b1_ref, w2_ref, b2_ref,
               w3_ref, b3_ref, out_ref):
    bb = item_ref.shape[0]
    f32 = jnp.float32
    age_oh = (age_ref[...] ==
              lax.broadcasted_iota(jnp.int32, (bb, AGE_ROWS_PAD), 1)).astype(f32)
    gen_oh = (gen_ref[...] ==
              lax.broadcasted_iota(jnp.int32, (bb, GEN_ROWS_PAD), 1)).astype(f32)
    age_emb = jnp.dot(age_oh, aget_ref[...], preferred_element_type=f32)
    gen_emb = jnp.dot(gen_oh, gent_ref[...], preferred_element_type=f32)
    h = (jnp.dot(item_ref[...], w1a_ref[...], preferred_element_type=f32)
         + jnp.dot(age_emb, w1b_ref[...], preferred_element_type=f32)
         + jnp.dot(gen_emb, w1c_ref[...], preferred_element_type=f32)
         + b1_ref[...])
    h = jnp.maximum(h, 0.0)
    h = jnp.maximum(jnp.dot(h, w2_ref[...], preferred_element_type=f32)
                    + b2_ref[...], 0.0)
    out_ref[...] = jnp.dot(h, w3_ref[...], preferred_element_type=f32) + b3_ref[...]


def kernel(query_itemid, query_item_age, query_item_gender,
           item_table, age_table, gender_table,
           W1, b1, W2, b2, W3, b3):
    item_emb = _gather_item_rows(item_table, query_itemid)

    age2 = query_item_age.astype(jnp.int32).reshape(BATCH, 1)
    gen2 = query_item_gender.astype(jnp.int32).reshape(BATCH, 1)
    aget = jnp.pad(age_table, ((0, AGE_ROWS_PAD - age_table.shape[0]), (0, 0)))
    gent = jnp.pad(gender_table, ((0, GEN_ROWS_PAD - gender_table.shape[0]), (0, 0)))
    w1a = W1[:D_ITEM]
    w1b = W1[D_ITEM:D_ITEM + D_AGE]
    w1c = W1[D_ITEM + D_AGE:]
    b1r = b1.reshape(1, -1)
    b2r = b2.reshape(1, -1)
    b3r = b3.reshape(1, -1)

    bb = 2048
    grid = (BATCH // bb,)
    const = lambda i: (0, 0)
    out = pl.pallas_call(
        _mlp_block,
        grid=grid,
        in_specs=[
            pl.BlockSpec((bb, D_ITEM), lambda i: (i, 0)),
            pl.BlockSpec((bb, 1), lambda i: (i, 0)),
            pl.BlockSpec((bb, 1), lambda i: (i, 0)),
            pl.BlockSpec((AGE_ROWS_PAD, D_AGE), const),
            pl.BlockSpec((GEN_ROWS_PAD, D_GENDER), const),
            pl.BlockSpec((D_ITEM, 256), const),
            pl.BlockSpec((D_AGE, 256), const),
            pl.BlockSpec((D_GENDER, 256), const),
            pl.BlockSpec((1, 256), const),
            pl.BlockSpec((256, 128), const),
            pl.BlockSpec((1, 128), const),
            pl.BlockSpec((128, 64), const),
            pl.BlockSpec((1, 64), const),
        ],
        out_specs=pl.BlockSpec((bb, 64), lambda i: (i, 0)),
        out_shape=jax.ShapeDtypeStruct((BATCH, 64), jnp.float32),
    )(item_emb, age2, gen2, aget, gent, w1a, w1b, w1c, b1r, W2, b2r, W3, b3r)
    return out
